# baseline (speedup 1.0000x reference)
import jax
import jax.numpy as jnp
from jax import lax
from jax.experimental import pallas as pl
from jax.experimental.pallas import tpu as pltpu
from jax.experimental.pallas import tpu_sc as plsc

_NC, _NS = 2, 16
_NW = _NC * _NS
_CHUNK_ROWS = 32


def _sc_copy_body(emb_hbm, out_hbm, buf0, buf1, sem0, sem1):
    seq, dim = out_hbm.shape
    rows_per_w = seq // _NW
    n_chunks = rows_per_w // _CHUNK_ROWS
    wid = lax.axis_index("s") * _NC + lax.axis_index("c")
    base = wid * rows_per_w
    bufs = (buf0, buf1)
    sems = (sem0, sem1)
    out_copies = [None] * n_chunks
    for i in range(n_chunks):
        b = bufs[i % 2]
        s = sems[i % 2]
        if i >= 2:
            out_copies[i - 2].wait()
        off = base + i * _CHUNK_ROWS
        cin = pltpu.make_async_copy(emb_hbm.at[pl.ds(off, _CHUNK_ROWS)], b, s)
        cin.start()
        cin.wait()
        cout = pltpu.make_async_copy(b, out_hbm.at[pl.ds(off, _CHUNK_ROWS)], s)
        cout.start()
        out_copies[i] = cout
    for i in range(max(n_chunks - 2, 0), n_chunks):
        out_copies[i].wait()


def kernel(x, emb):
    seq, dim = x.shape[1], emb.shape[1]
    mesh = plsc.VectorSubcoreMesh(core_axis_name="c", subcore_axis_name="s")
    k = pl.kernel(
        _sc_copy_body,
        out_type=jax.ShapeDtypeStruct((seq, dim), emb.dtype),
        mesh=mesh,
        scratch_types=[
            pltpu.VMEM((_CHUNK_ROWS, dim), emb.dtype),
            pltpu.VMEM((_CHUNK_ROWS, dim), emb.dtype),
            pltpu.SemaphoreType.DMA,
            pltpu.SemaphoreType.DMA,
        ],
    )
    return k(emb)


# speedup vs baseline: 1.0345x; 1.0345x over previous
import jax
import jax.numpy as jnp
from jax import lax
from jax.experimental import pallas as pl
from jax.experimental.pallas import tpu as pltpu
from jax.experimental.pallas import tpu_sc as plsc

_NC, _NS = 2, 16
_NW = _NC * _NS
_CHUNK_ROWS = 32
_NBUF = 3


def _sc_copy_body(emb_hbm, out_hbm, bufs, sems_in, sems_out):
    seq, dim = out_hbm.shape
    rows_per_w = seq // _NW
    n_chunks = rows_per_w // _CHUNK_ROWS
    wid = lax.axis_index("s") * _NC + lax.axis_index("c")
    base = wid * rows_per_w

    in_copies = [None] * n_chunks
    out_copies = [None] * n_chunks

    def start_in(i):
        b = i % _NBUF
        off = base + i * _CHUNK_ROWS
        c = pltpu.make_async_copy(
            emb_hbm.at[pl.ds(off, _CHUNK_ROWS)], bufs.at[b], sems_in.at[b])
        c.start()
        in_copies[i] = c

    def start_out(i):
        b = i % _NBUF
        off = base + i * _CHUNK_ROWS
        c = pltpu.make_async_copy(
            bufs.at[b], out_hbm.at[pl.ds(off, _CHUNK_ROWS)], sems_out.at[b])
        c.start()
        out_copies[i] = c

    for i in range(n_chunks + 1):
        if i < n_chunks:
            if i >= _NBUF:
                out_copies[i - _NBUF].wait()
            start_in(i)
        if i >= 1:
            in_copies[i - 1].wait()
            start_out(i - 1)
    for i in range(max(n_chunks - _NBUF, 0), n_chunks):
        out_copies[i].wait()


def kernel(x, emb):
    seq, dim = x.shape[1], emb.shape[1]
    mesh = plsc.VectorSubcoreMesh(core_axis_name="c", subcore_axis_name="s")
    k = pl.kernel(
        _sc_copy_body,
        out_type=jax.ShapeDtypeStruct((seq, dim), emb.dtype),
        mesh=mesh,
        scratch_types=[
            pltpu.VMEM((_NBUF, _CHUNK_ROWS, dim), emb.dtype),
            pltpu.SemaphoreType.DMA((_NBUF,)),
            pltpu.SemaphoreType.DMA((_NBUF,)),
        ],
    )
    return k(emb)
